# baseline (speedup 1.0000x reference)
import jax
import jax.numpy as jnp
from jax import lax
from jax.experimental import pallas as pl
from jax.experimental.pallas import tpu as pltpu, tpu_sc as plsc

VOCAB = 1000000
EMBED_DIM = 64
BATCH = 4096
SEQ = 200

NC = 2
NS = 16
NW = NC * NS

B = BATCH * SEQ
B_PER_W = B // NW
IDX_PER_STREAM = 128
STREAMS_PER_W = B_PER_W // IDX_PER_STREAM
GROUP_STREAMS = 4
GROUP_ROWS = GROUP_STREAMS * IDX_PER_STREAM
N_GROUPS = STREAMS_PER_W // GROUP_STREAMS


def _make_gather():
    mesh = plsc.VectorSubcoreMesh(core_axis_name="c", subcore_axis_name="s")

    @pl.kernel(
        out_type=jax.ShapeDtypeStruct((B, EMBED_DIM), jnp.float32),
        mesh=mesh,
        scratch_types=[
            pltpu.VMEM((STREAMS_PER_W, IDX_PER_STREAM), jnp.int32),
            pltpu.VMEM((GROUP_ROWS, EMBED_DIM), jnp.float32),
            pltpu.SemaphoreType.DMA,
        ],
        compiler_params=pltpu.CompilerParams(use_tc_tiling_on_sc=False),
    )
    def k(table_hbm, idx_hbm, out_hbm, idx_v, rows_v, sem):
        wid = lax.axis_index("s") * NC + lax.axis_index("c")
        pltpu.sync_copy(idx_hbm.at[wid], idx_v)
        out_base = wid * B_PER_W

        def group_body(g, _):
            copies = []
            for j in range(GROUP_STREAMS):
                copies.append(pltpu.async_copy(
                    table_hbm.at[idx_v.at[g * GROUP_STREAMS + j]],
                    rows_v.at[pl.ds(j * IDX_PER_STREAM, IDX_PER_STREAM)],
                    sem,
                ))
            for c in copies:
                c.wait()
            pltpu.sync_copy(rows_v, out_hbm.at[pl.ds(out_base + g * GROUP_ROWS, GROUP_ROWS)])
            return ()

        lax.fori_loop(0, N_GROUPS, group_body, (), unroll=False)

    return k


_gather = _make_gather()


def kernel(tokens, embed_weight):
    idx = tokens.reshape(-1).astype(jnp.int32).reshape(NW, STREAMS_PER_W, IDX_PER_STREAM)
    out = _gather(embed_weight, idx)
    return out.reshape(BATCH, SEQ, EMBED_DIM)


# speedup vs baseline: 1.0226x; 1.0226x over previous
import jax
import jax.numpy as jnp
from jax import lax
from jax.experimental import pallas as pl
from jax.experimental.pallas import tpu as pltpu, tpu_sc as plsc

VOCAB = 1000000
EMBED_DIM = 64
BATCH = 4096
SEQ = 200

NC = 2
NS = 16
NW = NC * NS

B = BATCH * SEQ
B_PER_W = B // NW
IDX_PER_STREAM = 128
STREAMS_PER_W = B_PER_W // IDX_PER_STREAM
GROUP_STREAMS = 4
GROUP_ROWS = GROUP_STREAMS * IDX_PER_STREAM
N_GROUPS = STREAMS_PER_W // GROUP_STREAMS


def _make_gather():
    mesh = plsc.VectorSubcoreMesh(core_axis_name="c", subcore_axis_name="s")

    @pl.kernel(
        out_type=jax.ShapeDtypeStruct((B, EMBED_DIM), jnp.float32),
        mesh=mesh,
        scratch_types=[
            pltpu.VMEM((STREAMS_PER_W, IDX_PER_STREAM), jnp.int32),
            pltpu.VMEM((GROUP_ROWS, EMBED_DIM), jnp.float32),
            pltpu.VMEM((GROUP_ROWS, EMBED_DIM), jnp.float32),
            pltpu.SemaphoreType.DMA,
            pltpu.SemaphoreType.DMA,
            pltpu.SemaphoreType.DMA,
        ],
        compiler_params=pltpu.CompilerParams(use_tc_tiling_on_sc=False),
    )
    def k(table_hbm, idx_hbm, out_hbm, idx_v, rows0, rows1, gsem, osem0, osem1):
        wid = lax.axis_index("s") * NC + lax.axis_index("c")
        pltpu.sync_copy(idx_hbm.at[wid], idx_v)
        out_base = wid * B_PER_W

        def out_slice(g):
            return out_hbm.at[pl.ds(out_base + g * GROUP_ROWS, GROUP_ROWS)]

        def gather_group(g, buf):
            copies = []
            for j in range(GROUP_STREAMS):
                copies.append(pltpu.async_copy(
                    table_hbm.at[idx_v.at[g * GROUP_STREAMS + j]],
                    buf.at[pl.ds(j * IDX_PER_STREAM, IDX_PER_STREAM)],
                    gsem,
                ))
            for c in copies:
                c.wait()

        def start_out(g, buf, osem):
            pltpu.async_copy(buf, out_slice(g), osem)

        def wait_out(g, buf, osem):
            pltpu.make_async_copy(buf, out_slice(g), osem).wait()

        gather_group(0, rows0)
        start_out(0, rows0, osem0)
        gather_group(1, rows1)
        start_out(1, rows1, osem1)

        def body(i, _):
            g0 = 2 + 2 * i
            wait_out(g0 - 2, rows0, osem0)
            gather_group(g0, rows0)
            start_out(g0, rows0, osem0)
            g1 = g0 + 1
            wait_out(g1 - 2, rows1, osem1)
            gather_group(g1, rows1)
            start_out(g1, rows1, osem1)
            return ()

        lax.fori_loop(0, (N_GROUPS - 2) // 2, body, (), unroll=False)

        wait_out(N_GROUPS - 2, rows0, osem0)
        wait_out(N_GROUPS - 1, rows1, osem1)

    return k


_gather = _make_gather()


def kernel(tokens, embed_weight):
    idx = tokens.reshape(-1).astype(jnp.int32).reshape(NW, STREAMS_PER_W, IDX_PER_STREAM)
    out = _gather(embed_weight, idx)
    return out.reshape(BATCH, SEQ, EMBED_DIM)


# speedup vs baseline: 1.0245x; 1.0019x over previous
import jax
import jax.numpy as jnp
from jax import lax
from jax.experimental import pallas as pl
from jax.experimental.pallas import tpu as pltpu, tpu_sc as plsc

VOCAB = 1000000
EMBED_DIM = 64
BATCH = 4096
SEQ = 200

NC = 2
NS = 16
NW = NC * NS

B = BATCH * SEQ
B_PER_W = B // NW
IDX_PER_STREAM = 128
STREAMS_PER_W = B_PER_W // IDX_PER_STREAM
GROUP_STREAMS = 5
GROUP_ROWS = GROUP_STREAMS * IDX_PER_STREAM
N_GROUPS = STREAMS_PER_W // GROUP_STREAMS


def _make_gather():
    mesh = plsc.VectorSubcoreMesh(core_axis_name="c", subcore_axis_name="s")

    @pl.kernel(
        out_type=jax.ShapeDtypeStruct((B, EMBED_DIM), jnp.float32),
        mesh=mesh,
        scratch_types=[
            pltpu.VMEM((STREAMS_PER_W, IDX_PER_STREAM), jnp.int32),
            pltpu.VMEM((GROUP_ROWS, EMBED_DIM), jnp.float32),
            pltpu.VMEM((GROUP_ROWS, EMBED_DIM), jnp.float32),
            pltpu.SemaphoreType.DMA,
            pltpu.SemaphoreType.DMA,
            pltpu.SemaphoreType.DMA,
        ],
        compiler_params=pltpu.CompilerParams(use_tc_tiling_on_sc=False),
    )
    def k(table_hbm, idx_hbm, out_hbm, idx_v, rows0, rows1, gsem, osem0, osem1):
        wid = lax.axis_index("s") * NC + lax.axis_index("c")
        pltpu.sync_copy(idx_hbm.at[wid], idx_v)
        out_base = wid * B_PER_W

        def gather_descr(g, j, buf):
            return pltpu.make_async_copy(
                table_hbm.at[idx_v.at[g * GROUP_STREAMS + j]],
                buf.at[pl.ds(j * IDX_PER_STREAM, IDX_PER_STREAM)],
                gsem,
            )

        def out_descr(g, buf, osem):
            return pltpu.make_async_copy(
                buf, out_hbm.at[pl.ds(out_base + g * GROUP_ROWS, GROUP_ROWS)], osem)

        def fire_gathers(g, buf):
            for j in range(GROUP_STREAMS):
                gather_descr(g, j, buf).start()

        def wait_gathers(g, buf):
            for j in range(GROUP_STREAMS):
                gather_descr(g, j, buf).wait()

        def drain(g, buf, osem):
            out_descr(g, buf, osem).start()
            out_descr(g, buf, osem).wait()

        fire_gathers(0, rows0)
        fire_gathers(1, rows1)

        def body(i, _):
            g0 = 2 * i
            wait_gathers(g0, rows0)
            drain(g0, rows0, osem0)
            fire_gathers(g0 + 2, rows0)
            g1 = g0 + 1
            wait_gathers(g1, rows1)
            drain(g1, rows1, osem1)
            fire_gathers(g1 + 2, rows1)
            return ()

        lax.fori_loop(0, N_GROUPS // 2 - 1, body, (), unroll=False)

        wait_gathers(N_GROUPS - 2, rows0)
        drain(N_GROUPS - 2, rows0, osem0)
        wait_gathers(N_GROUPS - 1, rows1)
        drain(N_GROUPS - 1, rows1, osem1)

    return k


_gather = _make_gather()


def kernel(tokens, embed_weight):
    idx = tokens.reshape(-1).astype(jnp.int32).reshape(NW, STREAMS_PER_W, IDX_PER_STREAM)
    out = _gather(embed_weight, idx)
    return out.reshape(BATCH, SEQ, EMBED_DIM)
